# baseline (speedup 1.0000x reference)
import functools

import jax
import jax.numpy as jnp
from jax import lax
from jax.experimental import pallas as pl
from jax.experimental.pallas import tpu as pltpu
from jax.experimental.pallas import tpu_sc as plsc

_LANES = 16


def _make_gather(B, L, V, D, n_workers, chunk_rows, nbuf):
    rows_per_w = B // n_workers
    nchunk = rows_per_w // chunk_rows
    cidx = chunk_rows * L
    assert rows_per_w % chunk_rows == 0
    assert D % _LANES == 0
    mesh = plsc.VectorSubcoreMesh(core_axis_name="c", subcore_axis_name="s")
    info = plsc.get_sparse_core_info()
    nc = info.num_cores

    @functools.partial(
        pl.kernel,
        mesh=mesh,
        out_type=jax.ShapeDtypeStruct((B, L, D), jnp.float32),
        scratch_types=[
            pltpu.VMEM((nchunk, cidx), jnp.int32),
            pltpu.VMEM((nbuf, cidx, D), jnp.float32),
            pltpu.VMEM((nbuf, chunk_rows, L, D), jnp.float32),
            pltpu.SemaphoreType.DMA((nchunk,)),
            pltpu.SemaphoreType.DMA((nbuf,)),
            pltpu.SemaphoreType.DMA((nbuf,)),
        ],
        compiler_params=pltpu.CompilerParams(use_tc_tiling_on_sc=False),
    )
    def gather_kernel(table_hbm, idx_hbm, out_hbm, idx_v, rows2_v, rows3_v,
                      idx_sem, gat_sem, out_sem):
        wid = lax.axis_index("s") * nc + lax.axis_index("c")
        base = wid * rows_per_w

        for j in range(nchunk):
            pltpu.make_async_copy(
                idx_hbm.at[pl.ds((base + j * chunk_rows) * L, cidx)],
                idx_v.at[j], idx_sem.at[j]).start()

        def gather_cp(j, slot):
            return pltpu.make_async_copy(
                table_hbm.at[idx_v.at[j]], rows2_v.at[slot], gat_sem.at[slot])

        def out_cp(j, slot):
            return pltpu.make_async_copy(
                rows3_v.at[slot],
                out_hbm.at[pl.ds(base + j * chunk_rows, chunk_rows)],
                out_sem.at[slot])

        def start_gather(j, slot):
            pltpu.make_async_copy(
                idx_hbm.at[pl.ds((base + j * chunk_rows) * L, cidx)],
                idx_v.at[j], idx_sem.at[j]).wait()
            gather_cp(j, slot).start()

        def finish_chunk(j, slot):
            gather_cp(j, slot).wait()

            def repack_row(q, _):
                for l in range(L):
                    for h in range(D // _LANES):
                        rows3_v[slot, q, l, pl.ds(h * _LANES, _LANES)] = (
                            rows2_v[slot, q * L + l, pl.ds(h * _LANES, _LANES)]
                        )
                return 0

            lax.fori_loop(0, chunk_rows, repack_row, 0, unroll=False)
            out_cp(j, slot).start()

        def body(j, _):
            slot = j % nbuf
            @pl.when(j >= nbuf)
            def _():
                out_cp(j - nbuf, slot).wait()

            start_gather(j, slot)

            @pl.when(j >= 1)
            def _():
                finish_chunk(j - 1, (j - 1) % nbuf)

            return 0

        lax.fori_loop(0, nchunk, body, 0, unroll=False)
        finish_chunk(nchunk - 1, (nchunk - 1) % nbuf)
        for s in range(nbuf):
            j_last = nchunk - nbuf + s
            out_cp(j_last, j_last % nbuf).wait()

    return gather_kernel


def kernel(word_indexes, W):
    B, L = word_indexes.shape
    V, D = W.shape
    idx = word_indexes.reshape(B * L)
    return _make_gather(B, L, V, D, n_workers=32, chunk_rows=32, nbuf=2)(
        W, idx)


# speedup vs baseline: 1.0815x; 1.0815x over previous
import functools

import jax
import jax.numpy as jnp
from jax import lax
from jax.experimental import pallas as pl
from jax.experimental.pallas import tpu as pltpu
from jax.experimental.pallas import tpu_sc as plsc


def _make_gather(B, L, V, D, n_workers, chunk_rows):
    rows_per_w = B // n_workers
    nchunk = rows_per_w // chunk_rows
    cidx = chunk_rows * L
    assert rows_per_w % chunk_rows == 0
    mesh = plsc.VectorSubcoreMesh(core_axis_name="c", subcore_axis_name="s")
    info = plsc.get_sparse_core_info()
    nc = info.num_cores

    @functools.partial(
        pl.kernel,
        mesh=mesh,
        out_type=jax.ShapeDtypeStruct((B, L, D), jnp.float32),
        scratch_types=[
            pltpu.VMEM((nchunk, cidx), jnp.int32),
            pltpu.VMEM((2, cidx, D), jnp.float32),
            pltpu.SemaphoreType.DMA((nchunk,)),
            pltpu.SemaphoreType.DMA((2,)),
            pltpu.SemaphoreType.DMA((2,)),
        ],
        compiler_params=pltpu.CompilerParams(use_tc_tiling_on_sc=False),
    )
    def gather_kernel(table_hbm, idx_hbm, out_hbm, idx_v, rows_v, idx_sem,
                      gat_sem, out_sem):
        wid = lax.axis_index("s") * nc + lax.axis_index("c")
        base = wid * rows_per_w

        for j in range(nchunk):
            pltpu.make_async_copy(
                idx_hbm.at[pl.ds((base + j * chunk_rows) * L, cidx)],
                idx_v.at[j], idx_sem.at[j]).start()

        def gather_cp(j, slot):
            return pltpu.make_async_copy(
                table_hbm.at[idx_v.at[j]], rows_v.at[slot], gat_sem.at[slot])

        def row_cp(j, slot, q):
            return pltpu.make_async_copy(
                rows_v.at[slot, pl.ds(q * L, L)],
                out_hbm.at[base + j * chunk_rows + q],
                out_sem.at[slot])

        def emit_chunk(j, slot):
            gather_cp(j, slot).wait()
            for q in range(chunk_rows):
                row_cp(j, slot, q).start()

        def drain_chunk(j, slot):
            for q in range(chunk_rows):
                row_cp(j, slot, q).wait()

        for j in range(nchunk):
            slot = j % 2
            if j >= 2:
                drain_chunk(j - 2, slot)
            pltpu.make_async_copy(
                idx_hbm.at[pl.ds((base + j * chunk_rows) * L, cidx)],
                idx_v.at[j], idx_sem.at[j]).wait()
            gather_cp(j, slot).start()
            if j >= 1:
                emit_chunk(j - 1, (j - 1) % 2)

        emit_chunk(nchunk - 1, (nchunk - 1) % 2)
        drain_chunk(nchunk - 2, (nchunk - 2) % 2)
        drain_chunk(nchunk - 1, (nchunk - 1) % 2)

    return gather_kernel


def kernel(word_indexes, W):
    B, L = word_indexes.shape
    V, D = W.shape
    idx = word_indexes.reshape(B * L)
    return _make_gather(B, L, V, D, n_workers=32, chunk_rows=64)(W, idx)


# speedup vs baseline: 1.0837x; 1.0020x over previous
import functools

import jax
import jax.numpy as jnp
from jax import lax
from jax.experimental import pallas as pl
from jax.experimental.pallas import tpu as pltpu
from jax.experimental.pallas import tpu_sc as plsc


def _make_gather(N, V, D, n_workers, chunk, nbuf=3):
    nchunk = N // (n_workers * chunk)
    b_per_w = N // n_workers
    mesh = plsc.VectorSubcoreMesh(core_axis_name="c", subcore_axis_name="s")
    info = plsc.get_sparse_core_info()
    nc = info.num_cores

    @functools.partial(
        pl.kernel,
        mesh=mesh,
        out_type=jax.ShapeDtypeStruct((N, D), jnp.float32),
        scratch_types=[
            pltpu.VMEM((nchunk, chunk), jnp.int32),
            pltpu.VMEM((nbuf, chunk, D), jnp.float32),
            pltpu.SemaphoreType.DMA((nchunk,)),
            pltpu.SemaphoreType.DMA((nbuf,)),
            pltpu.SemaphoreType.DMA((nbuf,)),
        ],
        compiler_params=pltpu.CompilerParams(use_tc_tiling_on_sc=False),
    )
    def gather_kernel(table_hbm, idx_hbm, out_hbm, idx_v, rows_v, idx_sem,
                      gat_sem, out_sem):
        wid = lax.axis_index("s") * nc + lax.axis_index("c")
        base = wid * b_per_w

        idx_cps = []
        for j in range(nchunk):
            cp = pltpu.make_async_copy(
                idx_hbm.at[pl.ds(base + j * chunk, chunk)], idx_v.at[j],
                idx_sem.at[j])
            cp.start()
            idx_cps.append(cp)

        gat_cps = [None] * nbuf
        out_cps = [None] * nbuf
        for j in range(nchunk):
            slot = j % nbuf
            if out_cps[slot] is not None:
                out_cps[slot].wait()
                out_cps[slot] = None
            idx_cps[j].wait()
            cp = pltpu.make_async_copy(
                table_hbm.at[idx_v.at[j]], rows_v.at[slot], gat_sem.at[slot])
            cp.start()
            gat_cps[slot] = cp
            prev = (j - 1) % nbuf
            if j >= 1 and gat_cps[prev] is not None:
                gat_cps[prev].wait()
                gat_cps[prev] = None
                ocp = pltpu.make_async_copy(
                    rows_v.at[prev],
                    out_hbm.at[pl.ds(base + (j - 1) * chunk, chunk)],
                    out_sem.at[prev])
                ocp.start()
                out_cps[prev] = ocp

        last = (nchunk - 1) % nbuf
        gat_cps[last].wait()
        ocp = pltpu.make_async_copy(
            rows_v.at[last],
            out_hbm.at[pl.ds(base + (nchunk - 1) * chunk, chunk)],
            out_sem.at[last])
        ocp.start()
        out_cps[last] = ocp
        for cp in out_cps:
            if cp is not None:
                cp.wait()

    return gather_kernel


def kernel(word_indexes, W):
    B, L = word_indexes.shape
    V, D = W.shape
    N = B * L
    idx = word_indexes.reshape(N).astype(jnp.int32)
    out = _make_gather(N, V, D, n_workers=32, chunk=1024)(W, idx)
    return out.reshape(B, L, D)
